# Initial kernel scaffold; baseline (speedup 1.0000x reference)
import jax
import jax.numpy as jnp
from jax.experimental import pallas as pl


def kernel(user_emb, item_emb, cat_emb, graph_rows, graph_cols, graph_vals, gx_rows, gx_cols, gx_vals, train_users, train_items, item_to_category, users, items):
    raise NotImplementedError("write your pallas kernel here")


# speedup vs baseline: 3.6363x; 3.6363x over previous
import functools

import jax
import jax.numpy as jnp
from jax import lax
from jax.experimental import pallas as pl
from jax.experimental.pallas import tpu as pltpu
from jax.experimental.pallas import tpu_sc as plsc

NC = 2
NS = 16
NW = NC * NS
L16 = 16
KB = 128
D = 128


def _mesh():
    return plsc.VectorSubcoreMesh(core_axis_name="c", subcore_axis_name="s",
                                  num_cores=NC, num_subcores=NS)


_CP = pltpu.CompilerParams(needs_layout_passes=False)


def _wid():
    return lax.axis_index("s") * NC + lax.axis_index("c")


def _bcast16(x):
    return jnp.broadcast_to(x, (L16,))


def _zero_fill(buf, rows):
    z = jnp.zeros((L16,), jnp.float32)

    def body(i, _):
        for j in range(D // L16):
            buf[i, pl.ds(j * L16, L16)] = z
        return 0

    lax.fori_loop(0, rows, body, 0)


def _zero_shared(acc, zb, np_rows):
    s = lax.axis_index("s")
    per_tile = np_rows // NS
    r0 = s * per_tile

    def body(t, _):
        pltpu.sync_copy(zb, acc.at[pl.ds(r0 + t * 64, 64)])
        return 0

    lax.fori_loop(0, per_tile // 64, body, 0)


def _flush_shared(acc, out, bnc, np_rows):
    c = lax.axis_index("c")
    s = lax.axis_index("s")
    per_tile = np_rows // NS
    r0 = s * per_tile

    def body(t, _):
        off = r0 + t * 64
        pltpu.sync_copy(acc.at[pl.ds(off, 64)], bnc)
        pltpu.sync_copy(bnc, out.at[c, pl.ds(off, 64)])
        return 0

    lax.fori_loop(0, per_tile // 64, body, 0)


def _nblocks(total_blocks):
    w = _wid()
    q, r = divmod(total_blocks, NW)
    return q + jnp.where(w < r, 1, 0).astype(jnp.int32)


def _prep_call(gx_rows, gx_cols, train_users, ti_off, itc_pad, np_rows):
    E = gx_rows.shape[0]
    nblk = E // KB
    ii = jnp.int32
    out_type = (
        jax.ShapeDtypeStruct((E,), ii),
        jax.ShapeDtypeStruct((E,), ii),
        jax.ShapeDtypeStruct((E,), ii),
        jax.ShapeDtypeStruct((E,), ii),
        jax.ShapeDtypeStruct((E,), ii),
        jax.ShapeDtypeStruct((NC, np_rows), jnp.float32),
    )

    @functools.partial(
        pl.kernel, out_type=out_type, mesh=_mesh(), compiler_params=_CP,
        scratch_types=[
            pltpu.VMEM((KB,), ii),
            pltpu.VMEM((KB,), ii),
            pltpu.VMEM((KB,), ii),
            pltpu.VMEM((KB,), ii),
            pltpu.VMEM((KB,), jnp.float32),
            pltpu.VMEM((np_rows // NS,), jnp.float32),
            pltpu.VMEM_SHARED((np_rows,), jnp.float32),
        ],
    )
    def prep(gxr, gxc, tu, tio, itc, srcu, srci, dstu, dsti, catx, degp,
             cb, rb, g1, g2, ones, dbnc, dacc):
        w = _wid()
        c = lax.axis_index("c")
        s = lax.axis_index("s")
        per_tile = np_rows // NS

        one = jnp.ones((L16,), jnp.float32)
        zero = jnp.zeros((L16,), jnp.float32)

        def fill(i, _):
            ones[pl.ds(i * L16, L16)] = one
            return 0
        lax.fori_loop(0, KB // L16, fill, 0)

        def zfill(i, _):
            dbnc[pl.ds(i * L16, L16)] = zero
            return 0
        lax.fori_loop(0, per_tile // L16, zfill, 0)
        pltpu.sync_copy(dbnc, dacc.at[pl.ds(s * per_tile, per_tile)])
        plsc.subcore_barrier()

        nb = _nblocks(nblk)

        def body(t, _):
            boff = (w + NW * t) * KB
            blk = pl.ds(boff, KB)
            pltpu.sync_copy(gxc.at[blk], cb)
            pltpu.sync_copy(gxr.at[blk], rb)
            pltpu.sync_copy(tu.at[cb], g1)
            pltpu.sync_copy(g1, srcu.at[blk])
            pltpu.sync_copy(tio.at[cb], g2)
            pltpu.sync_copy(g2, srci.at[blk])
            pltpu.sync_copy(itc.at[g2], g1)
            pltpu.sync_copy(g1, catx.at[blk])
            pltpu.sync_copy(tu.at[rb], g1)
            pltpu.sync_copy(g1, dstu.at[blk])
            pltpu.sync_copy(tio.at[rb], g2)
            pltpu.sync_copy(g2, dsti.at[blk])
            pltpu.sync_copy(tu.at[blk], cb)
            pltpu.sync_copy(ones, dacc.at[cb], add=True)
            pltpu.sync_copy(tio.at[blk], rb)
            pltpu.sync_copy(ones, dacc.at[rb], add=True)
            return 0

        lax.fori_loop(0, nb, body, 0)
        plsc.subcore_barrier()
        sl = pl.ds(s * per_tile, per_tile)
        pltpu.sync_copy(dacc.at[sl], dbnc)
        pltpu.sync_copy(dbnc, degp.at[c, sl])

    return prep(gx_rows, gx_cols, train_users, ti_off, itc_pad)


def _spmm_call(x, rows, cols, vals, np_rows):
    E2 = rows.shape[0]
    nblk = E2 // KB
    ii = jnp.int32
    out_type = jax.ShapeDtypeStruct((NC, np_rows, D), jnp.float32)

    @functools.partial(
        pl.kernel, out_type=out_type, mesh=_mesh(), compiler_params=_CP,
        scratch_types=[
            pltpu.VMEM((KB,), ii),
            pltpu.VMEM((KB,), ii),
            pltpu.VMEM((KB,), jnp.float32),
            pltpu.VMEM((KB, D), jnp.float32),
            pltpu.VMEM_SHARED((np_rows, D), jnp.float32),
        ],
    )
    def spmm(xr, rr, cr, vr, yp, cb, rb, vb, G, acc):
        w = _wid()
        _zero_fill(G, 64)
        _zero_shared(acc, G.at[pl.ds(0, 64)], np_rows)
        plsc.subcore_barrier()

        nb = _nblocks(nblk)

        def body(t, _):
            blk = pl.ds((w + NW * t) * KB, KB)
            pltpu.sync_copy(cr.at[blk], cb)
            pltpu.sync_copy(rr.at[blk], rb)
            pltpu.sync_copy(vr.at[blk], vb)
            pltpu.sync_copy(xr.at[cb], G)

            def scale(k, _c):
                wv = plsc.load_gather(vb, [_bcast16(k)])
                for j in range(D // L16):
                    sl = pl.ds(j * L16, L16)
                    G[k, sl] = G[k, sl] * wv
                return 0

            lax.fori_loop(0, KB, scale, 0)
            pltpu.sync_copy(G, acc.at[rb], add=True)
            return 0

        lax.fori_loop(0, nb, body, 0)
        plsc.subcore_barrier()
        _flush_shared(acc, yp, G.at[pl.ds(0, 64)], np_rows)

    return spmm(x, rows, cols, vals)


def _combine_call(yp, np_rows):
    out_type = jax.ShapeDtypeStruct((np_rows, D), jnp.float32)

    @functools.partial(
        pl.kernel, out_type=out_type, mesh=_mesh(), compiler_params=_CP,
        scratch_types=[
            pltpu.VMEM((64, D), jnp.float32),
            pltpu.VMEM((64, D), jnp.float32),
        ],
    )
    def combine(ypr, xo, b0, b1):
        w = _wid()
        per_w = np_rows // NW
        r0 = w * per_w

        def body(t, _):
            off = r0 + t * 64
            pltpu.sync_copy(ypr.at[0, pl.ds(off, 64)], b0)
            for c_ in range(1, NC):
                pltpu.sync_copy(ypr.at[c_, pl.ds(off, 64)], b1)

                def add(i, _c):
                    for j in range(D // L16):
                        sl = pl.ds(j * L16, L16)
                        b0[i, sl] = b0[i, sl] + b1[i, sl]
                    return 0

                lax.fori_loop(0, 64, add, 0)
            pltpu.sync_copy(b0, xo.at[pl.ds(off, 64)])
            return 0

        lax.fori_loop(0, per_w // 64, body, 0)

    return combine(yp)


def _gx_call(x, cat_emb, srcu, srci, dstu, dsti, catx, gxv, np_rows,
             with_cat):
    E = gxv.shape[0]
    kb = 64 if with_cat else KB
    nblk = E // kb
    ii = jnp.int32
    out_type = jax.ShapeDtypeStruct((NC, np_rows, D), jnp.float32)

    scratch = [
        pltpu.VMEM((kb,), ii),
        pltpu.VMEM((kb,), ii),
        pltpu.VMEM((kb,), ii),
        pltpu.VMEM((kb,), ii),
        pltpu.VMEM((kb,), jnp.float32),
        pltpu.VMEM((kb, D), jnp.float32),
        pltpu.VMEM((kb, D), jnp.float32),
        pltpu.VMEM_SHARED((np_rows, D), jnp.float32),
    ]
    if with_cat:
        scratch.insert(5, pltpu.VMEM((kb,), ii))
        scratch.append(pltpu.VMEM((kb, 2 * D), jnp.float32))

    @functools.partial(pl.kernel, out_type=out_type, mesh=_mesh(), compiler_params=_CP,
                       scratch_types=scratch)
    def gx(*args):
        if with_cat:
            (xr, cer, srcur, srcir, dstur, dstir, catxr, gxvr, up,
             su, si, du, di, wv, cx, GU, GI, acc, C) = args
        else:
            (xr, srcur, srcir, dstur, dstir, gxvr, up,
             su, si, du, di, wv, GU, GI, acc) = args
        w = _wid()
        _zero_fill(GU, 64)
        _zero_shared(acc, GU.at[pl.ds(0, 64)], np_rows)
        plsc.subcore_barrier()

        nb = _nblocks(nblk)

        def body(t, _):
            blk = pl.ds((w + NW * t) * kb, kb)
            pltpu.sync_copy(srcur.at[blk], su)
            pltpu.sync_copy(srcir.at[blk], si)
            pltpu.sync_copy(dstur.at[blk], du)
            pltpu.sync_copy(dstir.at[blk], di)
            pltpu.sync_copy(gxvr.at[blk], wv)
            pltpu.sync_copy(xr.at[su], GU)
            pltpu.sync_copy(xr.at[si], GI)
            if with_cat:
                pltpu.sync_copy(catxr.at[blk], cx)
                pltpu.sync_copy(cer.at[cx], C)

            def scale(k, _c):
                wvec = plsc.load_gather(wv, [_bcast16(k)])
                for j in range(D // L16):
                    sl = pl.ds(j * L16, L16)
                    gu = GU[k, sl]
                    gi = GI[k, sl]
                    if with_cat:
                        gu = gu + C[k, sl]
                        gi = gi + C[k, pl.ds(D + j * L16, L16)]
                    GU[k, sl] = gu * wvec
                    GI[k, sl] = gi * wvec
                return 0

            lax.fori_loop(0, kb, scale, 0)
            pltpu.sync_copy(GU, acc.at[du], add=True)
            pltpu.sync_copy(GI, acc.at[di], add=True)
            return 0

        lax.fori_loop(0, nb, body, 0)
        plsc.subcore_barrier()
        _flush_shared(acc, up, GU.at[pl.ds(0, 64)], np_rows)

    if with_cat:
        return gx(x, cat_emb, srcu, srci, dstu, dsti, catx, gxv)
    return gx(x, srcu, srci, dstu, dsti, gxv)


def _fin_call(up, degp, xc, np_rows):
    out_type = jax.ShapeDtypeStruct((np_rows, D), jnp.float32)

    @functools.partial(
        pl.kernel, out_type=out_type, mesh=_mesh(), compiler_params=_CP,
        scratch_types=[
            pltpu.VMEM((64, D), jnp.float32),
            pltpu.VMEM((64, D), jnp.float32),
            pltpu.VMEM((64, D), jnp.float32),
            pltpu.VMEM((64,), jnp.float32),
            pltpu.VMEM((64,), jnp.float32),
        ],
    )
    def fin(upr, degr, xcr, xo, u0, u1, xb, d0, d1):
        w = _wid()
        per_w = np_rows // NW
        r0 = w * per_w
        half = jnp.full((L16,), 0.5, jnp.float32)
        eps = jnp.full((L16,), 1e-9, jnp.float32)

        def body(t, _):
            off = r0 + t * 64
            sl64 = pl.ds(off, 64)
            pltpu.sync_copy(upr.at[0, sl64], u0)
            pltpu.sync_copy(xcr.at[sl64], xb)
            pltpu.sync_copy(degr.at[0, sl64], d0)
            for c_ in range(1, NC):
                pltpu.sync_copy(upr.at[c_, sl64], u1)
                pltpu.sync_copy(degr.at[c_, sl64], d1)

                def accp(i, _c):
                    for j in range(D // L16):
                        sl = pl.ds(j * L16, L16)
                        u0[i, sl] = u0[i, sl] + u1[i, sl]
                    return 0

                lax.fori_loop(0, 64, accp, 0)

                def accd(i, _c):
                    sl = pl.ds(i * L16, L16)
                    d0[sl] = d0[sl] + d1[sl]
                    return 0

                lax.fori_loop(0, 64 // L16, accd, 0)

            def recip(i, _c):
                sl = pl.ds(i * L16, L16)
                d0[sl] = jnp.float32(1.0) / (d0[sl] + eps)
                return 0

            lax.fori_loop(0, 64 // L16, recip, 0)

            def rowfix(k, _c):
                r = plsc.load_gather(d0, [_bcast16(k)])
                for j in range(D // L16):
                    sl = pl.ds(j * L16, L16)
                    xb[k, sl] = (u0[k, sl] * r + xb[k, sl]) * half
                return 0

            lax.fori_loop(0, 64, rowfix, 0)
            pltpu.sync_copy(xb, xo.at[sl64])
            return 0

        lax.fori_loop(0, per_w // 64, body, 0)

    return fin(up, degp, xc)


def _gamma_call(x1, x2, users, items_off):
    B = users.shape[0]
    pb = B // NW
    ngrp = pb // L16
    ii = jnp.int32
    out_type = jax.ShapeDtypeStruct((B,), jnp.float32)

    @functools.partial(
        pl.kernel, out_type=out_type, mesh=_mesh(), compiler_params=_CP,
        scratch_types=[
            pltpu.VMEM((pb,), ii),
            pltpu.VMEM((pb,), ii),
            pltpu.VMEM((pb, D), jnp.float32),
            pltpu.VMEM((pb, D), jnp.float32),
            pltpu.VMEM((pb, D), jnp.float32),
            pltpu.VMEM((pb, D), jnp.float32),
            pltpu.VMEM((pb,), jnp.float32),
        ],
    )
    def gam(x1r, x2r, ur, ir, go, ub, ib, U1, U2, I1, I2, ob):
        w = _wid()
        base = w * pb
        blk = pl.ds(base, pb)
        pltpu.sync_copy(ur.at[blk], ub)
        pltpu.sync_copy(ir.at[blk], ib)
        pltpu.sync_copy(x1r.at[ub], U1)
        pltpu.sync_copy(x2r.at[ub], U2)
        pltpu.sync_copy(x1r.at[ib], I1)
        pltpu.sync_copy(x2r.at[ib], I2)

        z = jnp.zeros((L16,), jnp.float32)
        for g in range(ngrp):
            ob[pl.ds(g * L16, L16)] = z
        iota = lax.iota(jnp.int32, L16)

        def body(t, _):
            g = t // D
            f = t % D
            rowi = g * L16 + iota
            coli = _bcast16(f)
            u = (plsc.load_gather(U1, [rowi, coli])
                 + plsc.load_gather(U2, [rowi, coli]))
            v = (plsc.load_gather(I1, [rowi, coli])
                 + plsc.load_gather(I2, [rowi, coli]))
            plsc.addupdate(ob.at[pl.ds(g * L16, L16)], u * v)
            return 0

        lax.fori_loop(0, ngrp * D, body, 0)
        q = jnp.full((L16,), 0.25, jnp.float32)
        for g in range(ngrp):
            sl = pl.ds(g * L16, L16)
            ob[sl] = ob[sl] * q
        pltpu.sync_copy(ob, go.at[blk])

    return gam(x1, x2, users, items_off)


def kernel(user_emb, item_emb, cat_emb, graph_rows, graph_cols, graph_vals,
           gx_rows, gx_cols, gx_vals, train_users, train_items,
           item_to_category, users, items):
    NU = user_emb.shape[0]
    NI = item_emb.shape[0]
    N = NU + NI
    np_rows = ((N + 1023) // 1024) * 1024

    ii = jnp.int32
    x0 = jnp.concatenate([user_emb, item_emb], axis=0)
    x0 = jnp.pad(x0, ((0, np_rows - N), (0, 0)))
    ti_off = (train_items + NU).astype(ii)
    itc_pad = jnp.concatenate(
        [jnp.zeros((NU,), ii), item_to_category.astype(ii)])
    items_off = (items + NU).astype(ii)

    srcu, srci, dstu, dsti, catx, degp = _prep_call(
        gx_rows.astype(ii), gx_cols.astype(ii), train_users.astype(ii),
        ti_off, itc_pad, np_rows)

    x = x0
    layer_out = []
    for layer in range(2):
        yp = _spmm_call(x, graph_rows.astype(ii), graph_cols.astype(ii),
                        graph_vals, np_rows)
        y = _combine_call(yp, np_rows)
        up = _gx_call(y, cat_emb, srcu, srci, dstu, dsti, catx, gx_vals,
                      np_rows, with_cat=(layer == 0))
        x = _fin_call(up, degp, y, np_rows)
        layer_out.append(x)

    return _gamma_call(layer_out[0], layer_out[1], users.astype(ii),
                       items_off)
